# baseline (speedup 1.0000x reference)
import functools

import jax
import jax.numpy as jnp
from jax import lax
from jax.experimental import pallas as pl
from jax.experimental.pallas import tpu as pltpu
from jax.experimental.pallas import tpu_sc as plsc

_SLOT_ROWS = 256
_NBUF = 4


@functools.lru_cache(maxsize=None)
def _build(seq_len: int, channels: int, dtype_name: str):
    info = plsc.get_sparse_core_info()
    nc = info.num_cores
    assert seq_len % (nc * _SLOT_ROWS) == 0
    rows_per_w = seq_len // nc
    nchunks = rows_per_w // _SLOT_ROWS
    dtype = jnp.dtype(dtype_name)
    mesh = plsc.ScalarSubcoreMesh(axis_name="c")

    def body(w_hbm, out_hbm, spbuf, load_sem, store_sem):
        cid = lax.axis_index("c")
        base = cid * rows_per_w

        def load(i, b):
            return pltpu.async_copy(
                w_hbm.at[pl.ds(base + i * _SLOT_ROWS, _SLOT_ROWS)],
                spbuf.at[pl.ds(b * _SLOT_ROWS, _SLOT_ROWS)],
                load_sem.at[b])

        def store(i, b):
            return pltpu.async_copy(
                spbuf.at[pl.ds(b * _SLOT_ROWS, _SLOT_ROWS)],
                out_hbm.at[pl.ds(base + i * _SLOT_ROWS, _SLOT_ROWS)],
                store_sem.at[b])

        loads = [None] * _NBUF
        stores = [None] * _NBUF
        loads[0] = load(0, 0)
        for i in range(nchunks):
            b = i % _NBUF
            nb = (i + 1) % _NBUF
            if i + 1 < nchunks:
                if stores[nb] is not None:
                    stores[nb].wait()
                loads[nb] = load(i + 1, nb)
            loads[b].wait()
            stores[b] = store(i, b)
        for b in range(_NBUF):
            if stores[b] is not None:
                stores[b].wait()

    return pl.kernel(
        body,
        out_type=jax.ShapeDtypeStruct((seq_len, channels), dtype),
        mesh=mesh,
        scratch_types=[
            pltpu.VMEM_SHARED((_NBUF * _SLOT_ROWS, channels), dtype),
            pltpu.SemaphoreType.DMA((_NBUF,)),
            pltpu.SemaphoreType.DMA((_NBUF,)),
        ],
    )


def kernel(x, W):
    seq_len = x.shape[1]
    k = _build(seq_len, W.shape[1], W.dtype.name)
    return k(W)


# speedup vs baseline: 1.0352x; 1.0352x over previous
import functools

import jax
import jax.numpy as jnp
from jax import lax
from jax.experimental import pallas as pl
from jax.experimental.pallas import tpu as pltpu
from jax.experimental.pallas import tpu_sc as plsc

_A_TILES = 8
_A_BUF_ROWS = 40
_B_SLOT_ROWS = 32
_NBUF = 2


def _ring(load, store, nchunks):
    loads = [None] * _NBUF
    stores = [None] * _NBUF
    loads[0] = load(0, 0)
    for i in range(nchunks):
        b = i % _NBUF
        nb = (i + 1) % _NBUF
        if i + 1 < nchunks:
            if stores[nb] is not None:
                stores[nb].wait()
            loads[nb] = load(i + 1, nb)
        loads[b].wait()
        stores[b] = store(i, b)
    for b in range(_NBUF):
        if stores[b] is not None:
            stores[b].wait()


@functools.lru_cache(maxsize=None)
def _build(seq_len: int, channels: int, dtype_name: str):
    info = plsc.get_sparse_core_info()
    nc = info.num_cores
    ns = info.num_subcores
    b_tiles = ns - _A_TILES
    half = seq_len // 2
    na = nc * _A_TILES
    nb_w = nc * b_tiles
    assert half % na == 0 and half % (nb_w * _B_SLOT_ROWS) == 0
    rows_a = half // na
    rows_b = half // nb_w
    sizes = []
    left = rows_a
    while left > 0:
        c = min(_A_BUF_ROWS, left)
        sizes.append(c)
        left -= c
    starts = [sum(sizes[:i]) for i in range(len(sizes))]
    nchunks_b = rows_b // _B_SLOT_ROWS
    dtype = jnp.dtype(dtype_name)
    mesh = plsc.VectorSubcoreMesh(core_axis_name="c", subcore_axis_name="s")

    def body(w_hbm, out_hbm, tbuf, spbuf, load_sem, store_sem):
        cid = lax.axis_index("c")
        sid = lax.axis_index("s")

        @pl.when(sid < _A_TILES)
        def _path_a():
            wid = cid * _A_TILES + sid
            base = wid * rows_a

            def load(i, b):
                return pltpu.async_copy(
                    w_hbm.at[pl.ds(base + starts[i], sizes[i])],
                    tbuf.at[pl.ds(b * _A_BUF_ROWS, sizes[i])],
                    load_sem.at[b])

            def store(i, b):
                return pltpu.async_copy(
                    tbuf.at[pl.ds(b * _A_BUF_ROWS, sizes[i])],
                    out_hbm.at[pl.ds(base + starts[i], sizes[i])],
                    store_sem.at[b])

            _ring(load, store, len(sizes))

        @pl.when(sid >= _A_TILES)
        def _path_b():
            wid = cid * b_tiles + (sid - _A_TILES)
            base = half + wid * rows_b
            sp0 = (sid - _A_TILES) * (_NBUF * _B_SLOT_ROWS)

            def load(i, b):
                return pltpu.async_copy(
                    w_hbm.at[pl.ds(base + i * _B_SLOT_ROWS, _B_SLOT_ROWS)],
                    spbuf.at[pl.ds(sp0 + b * _B_SLOT_ROWS, _B_SLOT_ROWS)],
                    load_sem.at[b])

            def store(i, b):
                return pltpu.async_copy(
                    spbuf.at[pl.ds(sp0 + b * _B_SLOT_ROWS, _B_SLOT_ROWS)],
                    out_hbm.at[pl.ds(base + i * _B_SLOT_ROWS, _B_SLOT_ROWS)],
                    store_sem.at[b])

            _ring(load, store, nchunks_b)

    return pl.kernel(
        body,
        out_type=jax.ShapeDtypeStruct((seq_len, channels), dtype),
        mesh=mesh,
        scratch_types=[
            pltpu.VMEM((_NBUF * _A_BUF_ROWS, channels), dtype),
            pltpu.VMEM_SHARED(
                ((16 - _A_TILES) * _NBUF * _B_SLOT_ROWS, channels), dtype),
            pltpu.SemaphoreType.DMA((_NBUF,)),
            pltpu.SemaphoreType.DMA((_NBUF,)),
        ],
    )


def kernel(x, W):
    seq_len = x.shape[1]
    k = _build(seq_len, W.shape[1], W.dtype.name)
    return k(W)


# speedup vs baseline: 1.0603x; 1.0242x over previous
import functools

import jax
import jax.numpy as jnp
from jax import lax
from jax.experimental import pallas as pl
from jax.experimental.pallas import tpu as pltpu
from jax.experimental.pallas import tpu_sc as plsc

_BUF_ROWS = 56
_NBUF = 2


@functools.lru_cache(maxsize=None)
def _build(seq_len: int, channels: int, dtype_name: str):
    info = plsc.get_sparse_core_info()
    nw = info.num_cores * info.num_subcores
    assert seq_len % nw == 0
    rows_per_w = seq_len // nw
    sizes = []
    left = rows_per_w
    while left > 0:
        c = min(_BUF_ROWS, left)
        sizes.append(c)
        left -= c
    starts = [sum(sizes[:i]) for i in range(len(sizes))]
    nchunks = len(sizes)
    dtype = jnp.dtype(dtype_name)
    mesh = plsc.VectorSubcoreMesh(core_axis_name="c", subcore_axis_name="s")

    def body(w_hbm, out_hbm, buf, load_sem, store_sem):
        wid = lax.axis_index("s") * info.num_cores + lax.axis_index("c")
        base = wid * rows_per_w

        def load(i, b):
            return pltpu.async_copy(
                w_hbm.at[pl.ds(base + starts[i], sizes[i])],
                buf.at[pl.ds(b * _BUF_ROWS, sizes[i])],
                load_sem.at[b])

        def store(i, b):
            return pltpu.async_copy(
                buf.at[pl.ds(b * _BUF_ROWS, sizes[i])],
                out_hbm.at[pl.ds(base + starts[i], sizes[i])],
                store_sem.at[b])

        loads = [None] * _NBUF
        stores = [None] * _NBUF
        loads[0] = load(0, 0)
        for i in range(nchunks):
            b = i % _NBUF
            nb = (i + 1) % _NBUF
            if i + 1 < nchunks:
                if stores[nb] is not None:
                    stores[nb].wait()
                loads[nb] = load(i + 1, nb)
            loads[b].wait()
            stores[b] = store(i, b)
        for b in range(_NBUF):
            if stores[b] is not None:
                stores[b].wait()

    return pl.kernel(
        body,
        out_type=jax.ShapeDtypeStruct((seq_len, channels), dtype),
        mesh=mesh,
        scratch_types=[
            pltpu.VMEM((_NBUF * _BUF_ROWS, channels), dtype),
            pltpu.SemaphoreType.DMA((_NBUF,)),
            pltpu.SemaphoreType.DMA((_NBUF,)),
        ],
    )


def kernel(x, W):
    seq_len = x.shape[1]
    k = _build(seq_len, W.shape[1], W.dtype.name)
    return k(W)
